# baseline (speedup 1.0000x reference)
import functools

import jax
import jax.numpy as jnp
from jax import lax
from jax.experimental import pallas as pl
from jax.experimental.pallas import tpu as pltpu
from jax.experimental.pallas import tpu_sc as plsc

N = 160000
D_IN = 256
D_H = 512
N_SEG = 1024


TILE = 8000
NBLK = N // TILE
SPLIT_BLK = 12
H1 = SPLIT_BLK * TILE
H2 = N - H1


def _mlp_body(px_ref, w1_ref, b1_ref, w2_ref, b2_ref, w3_ref, y_ref):
    def _silu_half(t):
        return t + t * jnp.tanh(t)

    def _chain(x):
        t = jnp.dot(x, w1_ref[...], preferred_element_type=jnp.float32)
        h = _silu_half(t + b1_ref[...])
        t = jnp.dot(h, w2_ref[...], preferred_element_type=jnp.float32)
        h = _silu_half(t + b2_ref[...])
        return jnp.dot(h, w3_ref[...], preferred_element_type=jnp.float32)

    x = px_ref[...]
    q = TILE // 4
    for i in range(4):
        y_ref[0, i * q:(i + 1) * q, :] = _chain(x[i * q:(i + 1) * q, :])


@functools.cache
def _mlp(nblk, blk_off):
    return pl.pallas_call(
        _mlp_body,
        grid=(nblk,),
        in_specs=[
            pl.BlockSpec((TILE, D_IN), lambda i: (i + blk_off, 0)),
            pl.BlockSpec((D_IN, D_H), lambda i: (0, 0)),
            pl.BlockSpec((1, D_H), lambda i: (0, 0)),
            pl.BlockSpec((D_H, D_H), lambda i: (0, 0)),
            pl.BlockSpec((1, D_H), lambda i: (0, 0)),
            pl.BlockSpec((D_H, 1), lambda i: (0, 0)),
        ],
        out_specs=pl.BlockSpec((1, TILE, 1), lambda i: (i, 0, 0)),
        out_shape=jax.ShapeDtypeStruct((nblk, TILE, 1), jnp.float32),
        compiler_params=pltpu.CompilerParams(
            dimension_semantics=("parallel",),
        ),
    )


NC = 2
NS = 16
LANES = 16
NW = NC * NS
SEG_CH = N_SEG // LANES


def _make_segsum_body(chunk, ids_off, nvec, tail):
    def _segsum_body(y_hbm, ids_hbm, out_hbm, yv, idv, acc, part):
        c = lax.axis_index("c")
        s = lax.axis_index("s")
        wid = s * NC + c
        ybase = wid * chunk
        pltpu.sync_copy(y_hbm.at[pl.ds(ybase, chunk)], yv.at[pl.ds(0, chunk)])
        pltpu.sync_copy(
            ids_hbm.at[pl.ds(ids_off + ybase, chunk)], idv.at[pl.ds(0, chunk)]
        )

        lane = lax.iota(jnp.int32, LANES)
        zeros16 = jnp.zeros((LANES,), jnp.float32)

        def _zero(i, carry):
            for l in range(LANES):
                acc[l, pl.ds(i * LANES, LANES)] = zeros16
            return carry

        lax.fori_loop(0, SEG_CH, _zero, 0)

        def _scat(i, carry):
            vals = yv[pl.ds(i * LANES, LANES)]
            ids = idv[pl.ds(i * LANES, LANES)]
            plsc.addupdate_scatter(acc, [lane, ids], vals)
            return carry

        lax.fori_loop(0, nvec, _scat, 0)

        if tail:
            tvals = yv[pl.ds(nvec * LANES, LANES)]
            tids = idv[pl.ds(nvec * LANES, LANES)]
            plsc.addupdate_scatter(acc, [lane, tids], tvals, mask=lane < tail)

        def _comb(j, carry):
            tot = acc[0, pl.ds(j * LANES, LANES)]
            for l in range(1, LANES):
                tot = tot + acc[l, pl.ds(j * LANES, LANES)]
            part[j, :] = tot
            return carry

        lax.fori_loop(0, SEG_CH, _comb, 0)

        pltpu.sync_copy(part, out_hbm.at[wid])

    return _segsum_body


@functools.cache
def _segsum(n_atoms, ids_off):
    chunk = n_atoms // NW
    assert chunk * NW == n_atoms and chunk % 8 == 0 and (ids_off + chunk) % 8 == 0
    nvec = chunk // LANES
    tail = chunk - nvec * LANES
    buf = (nvec + (1 if tail else 0)) * LANES
    mesh = plsc.VectorSubcoreMesh(core_axis_name="c", subcore_axis_name="s")
    return pl.kernel(
        _make_segsum_body(chunk, ids_off, nvec, tail),
        mesh=mesh,
        compiler_params=pltpu.CompilerParams(
            use_tc_tiling_on_sc=False, needs_layout_passes=False
        ),
        out_type=jax.ShapeDtypeStruct((NW, SEG_CH, LANES), jnp.float32),
        scratch_types=[
            pltpu.VMEM((buf,), jnp.float32),
            pltpu.VMEM((buf,), jnp.int32),
            pltpu.VMEM((LANES, N_SEG), jnp.float32),
            pltpu.VMEM((SEG_CH, LANES), jnp.float32),
        ],
    )


def kernel(atom_batch, px, W1, b1, W2, b2, W3):
    b1r = (0.5 * b1).reshape(1, D_H)
    b2r = (0.5 * b2).reshape(1, D_H)
    w1h = 0.5 * W1
    w2h = 0.5 * W2
    y1 = _mlp(SPLIT_BLK, 0)(px, w1h, b1r, w2h, b2r, W3).reshape(H1)
    y2 = _mlp(NBLK - SPLIT_BLK, SPLIT_BLK)(px, w1h, b1r, w2h, b2r, W3).reshape(H2)
    parts1 = _segsum(H1, 0)(y1, atom_batch)
    parts2 = _segsum(H2, H1)(y2, atom_batch)
    return (parts1.sum(axis=0) + parts2.sum(axis=0)).reshape(N_SEG)


# speedup vs baseline: 1.0647x; 1.0647x over previous
import functools

import jax
import jax.numpy as jnp
from jax import lax
from jax.experimental import pallas as pl
from jax.experimental.pallas import tpu as pltpu
from jax.experimental.pallas import tpu_sc as plsc

N = 160000
D_IN = 256
D_H = 512
N_SEG = 1024


TILE = 16000
NBLK = N // TILE


def _mlp_body(px_ref, w1_ref, b1_ref, w2_ref, b2_ref, w3_ref, y_ref):
    def _silu_half(t):
        return t + t * jnp.tanh(t)

    def _chain(x):
        t = jnp.dot(x, w1_ref[...], preferred_element_type=jnp.float32)
        h = _silu_half(t + b1_ref[...])
        t = jnp.dot(h, w2_ref[...], preferred_element_type=jnp.float32)
        h = _silu_half(t + b2_ref[...])
        return jnp.dot(h, w3_ref[...], preferred_element_type=jnp.float32)

    x = px_ref[...]
    q = TILE // 8
    for i in range(8):
        y_ref[0, i * q:(i + 1) * q, :] = _chain(x[i * q:(i + 1) * q, :])


def _mlp(px, W1, b1r, W2, b2r, w3r):
    return pl.pallas_call(
        _mlp_body,
        grid=(NBLK,),
        in_specs=[
            pl.BlockSpec((TILE, D_IN), lambda i: (i, 0)),
            pl.BlockSpec((D_IN, D_H), lambda i: (0, 0)),
            pl.BlockSpec((1, D_H), lambda i: (0, 0)),
            pl.BlockSpec((D_H, D_H), lambda i: (0, 0)),
            pl.BlockSpec((1, D_H), lambda i: (0, 0)),
            pl.BlockSpec((D_H, 1), lambda i: (0, 0)),
        ],
        out_specs=pl.BlockSpec((1, TILE, 1), lambda i: (i, 0, 0)),
        out_shape=jax.ShapeDtypeStruct((NBLK, TILE, 1), jnp.float32),
        compiler_params=pltpu.CompilerParams(
            dimension_semantics=("parallel",),
        ),
    )(px, W1, b1r, W2, b2r, w3r)


NC = 2
NS = 16
LANES = 16
NW = NC * NS
CHUNK = N // NW
NVEC = CHUNK // LANES
TAIL = CHUNK - NVEC * LANES
BUF = (NVEC + 1) * LANES
SEG_CH = N_SEG // LANES

def _segsum_body(y_hbm, ids_hbm, out_hbm, yv, idv, acc, part):
    c = lax.axis_index("c")
    s = lax.axis_index("s")
    wid = s * NC + c
    base = wid * CHUNK
    pltpu.sync_copy(y_hbm.at[pl.ds(base, CHUNK)], yv.at[pl.ds(0, CHUNK)])
    pltpu.sync_copy(ids_hbm.at[pl.ds(base, CHUNK)], idv.at[pl.ds(0, CHUNK)])

    lane = lax.iota(jnp.int32, LANES)
    zeros16 = jnp.zeros((LANES,), jnp.float32)

    def _zero(i, carry):
        for l in range(LANES):
            acc[l, pl.ds(i * LANES, LANES)] = zeros16
        return carry

    lax.fori_loop(0, SEG_CH, _zero, 0)

    def _scat(i, carry):
        vals = yv[pl.ds(i * LANES, LANES)]
        ids = idv[pl.ds(i * LANES, LANES)]
        plsc.addupdate_scatter(acc, [lane, ids], vals)
        return carry

    lax.fori_loop(0, NVEC, _scat, 0)

    tvals = yv[pl.ds(NVEC * LANES, LANES)]
    tids = idv[pl.ds(NVEC * LANES, LANES)]
    plsc.addupdate_scatter(acc, [lane, tids], tvals, mask=lane < TAIL)

    def _comb(j, carry):
        tot = acc[0, pl.ds(j * LANES, LANES)]
        for l in range(1, LANES):
            tot = tot + acc[l, pl.ds(j * LANES, LANES)]
        part[j, :] = tot
        return carry

    lax.fori_loop(0, SEG_CH, _comb, 0)

    pltpu.sync_copy(part, out_hbm.at[wid])


@functools.cache
def _segsum():
    mesh = plsc.VectorSubcoreMesh(core_axis_name="c", subcore_axis_name="s")
    return pl.kernel(
        _segsum_body,
        mesh=mesh,
        compiler_params=pltpu.CompilerParams(
            use_tc_tiling_on_sc=False, needs_layout_passes=False
        ),
        out_type=jax.ShapeDtypeStruct((NW, SEG_CH, LANES), jnp.float32),
        scratch_types=[
            pltpu.VMEM((BUF,), jnp.float32),
            pltpu.VMEM((BUF,), jnp.int32),
            pltpu.VMEM((LANES, N_SEG), jnp.float32),
            pltpu.VMEM((SEG_CH, LANES), jnp.float32),
        ],
    )


def kernel(atom_batch, px, W1, b1, W2, b2, W3):
    b1r = (0.5 * b1).reshape(1, D_H)
    b2r = (0.5 * b2).reshape(1, D_H)
    y = _mlp(px, 0.5 * W1, b1r, 0.5 * W2, b2r, W3).reshape(N)
    parts = _segsum()(y, atom_batch)
    return parts.sum(axis=0).reshape(N_SEG)


# speedup vs baseline: 1.0937x; 1.0272x over previous
import functools

import jax
import jax.numpy as jnp
from jax import lax
from jax.experimental import pallas as pl
from jax.experimental.pallas import tpu as pltpu
from jax.experimental.pallas import tpu_sc as plsc

N = 160000
D_IN = 256
D_H = 512
N_SEG = 1024


TILE = 8000
NBLK = N // TILE


def _mlp_body(px_ref, w1_ref, b1_ref, w2_ref, b2_ref, w3_ref, y_ref,
              w1s, b1s, w2s, b2s):
    @pl.when(pl.program_id(0) == 0)
    def _scale_weights():
        w1s[...] = w1_ref[...] * 0.5
        b1s[...] = b1_ref[...] * 0.5
        w2s[...] = w2_ref[...] * 0.5
        b2s[...] = b2_ref[...] * 0.5

    def _silu_half(t):
        return t + t * jnp.tanh(t)

    def _chain(x):
        t = jnp.dot(x, w1s[...], preferred_element_type=jnp.float32)
        h = _silu_half(t + b1s[...])
        t = jnp.dot(h, w2s[...], preferred_element_type=jnp.float32)
        h = _silu_half(t + b2s[...])
        return jnp.dot(h, w3_ref[...], preferred_element_type=jnp.float32)

    x = px_ref[...]
    q = TILE // 4
    for i in range(4):
        y_ref[0, i * q:(i + 1) * q, :] = _chain(x[i * q:(i + 1) * q, :])


def _mlp(px, W1, b1r, W2, b2r, w3r):
    return pl.pallas_call(
        _mlp_body,
        grid=(NBLK,),
        in_specs=[
            pl.BlockSpec((TILE, D_IN), lambda i: (i, 0)),
            pl.BlockSpec((D_IN, D_H), lambda i: (0, 0)),
            pl.BlockSpec((1, D_H), lambda i: (0, 0)),
            pl.BlockSpec((D_H, D_H), lambda i: (0, 0)),
            pl.BlockSpec((1, D_H), lambda i: (0, 0)),
            pl.BlockSpec((D_H, 1), lambda i: (0, 0)),
        ],
        out_specs=pl.BlockSpec((1, TILE, 1), lambda i: (i, 0, 0)),
        out_shape=jax.ShapeDtypeStruct((NBLK, TILE, 1), jnp.float32),
        scratch_shapes=[
            pltpu.VMEM((D_IN, D_H), jnp.float32),
            pltpu.VMEM((1, D_H), jnp.float32),
            pltpu.VMEM((D_H, D_H), jnp.float32),
            pltpu.VMEM((1, D_H), jnp.float32),
        ],
        compiler_params=pltpu.CompilerParams(
            dimension_semantics=("arbitrary",),
        ),
    )(px, W1, b1r, W2, b2r, w3r)


NC = 2
NS = 16
LANES = 16
NW = NC * NS
CHUNK = N // NW
NVEC = CHUNK // LANES
TAIL = CHUNK - NVEC * LANES
BUF = (NVEC + 1) * LANES
SEG_CH = N_SEG // LANES

def _segsum_body(y_hbm, ids_hbm, out_hbm, yv, idv, acc, part):
    c = lax.axis_index("c")
    s = lax.axis_index("s")
    wid = s * NC + c
    base = wid * CHUNK
    pltpu.sync_copy(y_hbm.at[pl.ds(base, CHUNK)], yv.at[pl.ds(0, CHUNK)])
    pltpu.sync_copy(ids_hbm.at[pl.ds(base, CHUNK)], idv.at[pl.ds(0, CHUNK)])

    lane = lax.iota(jnp.int32, LANES)
    zeros16 = jnp.zeros((LANES,), jnp.float32)

    def _zero(i, carry):
        for l in range(LANES):
            acc[l, pl.ds(i * LANES, LANES)] = zeros16
        return carry

    lax.fori_loop(0, SEG_CH, _zero, 0)

    def _scat(i, carry):
        vals = yv[pl.ds(i * LANES, LANES)]
        ids = idv[pl.ds(i * LANES, LANES)]
        plsc.addupdate_scatter(acc, [lane, ids], vals)
        return carry

    lax.fori_loop(0, NVEC, _scat, 0)

    tvals = yv[pl.ds(NVEC * LANES, LANES)]
    tids = idv[pl.ds(NVEC * LANES, LANES)]
    plsc.addupdate_scatter(acc, [lane, tids], tvals, mask=lane < TAIL)

    def _comb(j, carry):
        tot = acc[0, pl.ds(j * LANES, LANES)]
        for l in range(1, LANES):
            tot = tot + acc[l, pl.ds(j * LANES, LANES)]
        part[j, :] = tot
        return carry

    lax.fori_loop(0, SEG_CH, _comb, 0)

    pltpu.sync_copy(part, out_hbm.at[wid])


@functools.cache
def _segsum():
    mesh = plsc.VectorSubcoreMesh(core_axis_name="c", subcore_axis_name="s")
    return pl.kernel(
        _segsum_body,
        mesh=mesh,
        compiler_params=pltpu.CompilerParams(
            use_tc_tiling_on_sc=False, needs_layout_passes=False
        ),
        out_type=jax.ShapeDtypeStruct((NW, SEG_CH, LANES), jnp.float32),
        scratch_types=[
            pltpu.VMEM((BUF,), jnp.float32),
            pltpu.VMEM((BUF,), jnp.int32),
            pltpu.VMEM((LANES, N_SEG), jnp.float32),
            pltpu.VMEM((SEG_CH, LANES), jnp.float32),
        ],
    )


def kernel(atom_batch, px, W1, b1, W2, b2, W3):
    y = _mlp(px, W1, b1.reshape(1, D_H), W2, b2.reshape(1, D_H), W3).reshape(N)
    parts = _segsum()(y, atom_batch)
    return parts.sum(axis=0).reshape(N_SEG)
